# baseline (speedup 1.0000x reference)
import jax
import jax.numpy as jnp
from jax import lax
from jax.experimental import pallas as pl
from jax.experimental.pallas import tpu as pltpu
from jax.experimental.pallas import tpu_sc as plsc

_B = 16384
_D = 16
_NC = 2
_NS = 16
_NW = _NC * _NS
_BW = _B // _NW
_G = _BW // 16


def _pairwise_body(u_idx_hbm, p_idx_hbm, n_idx_hbm, user_t_hbm, item_t_hbm,
                   out_hbm, uidx_v, pidx_v, nidx_v, u_v, i_v, j_v, out_v,
                   sem):
    wid = lax.axis_index("s") * _NC + lax.axis_index("c")
    base = wid * _BW
    pltpu.sync_copy(u_idx_hbm.at[pl.ds(base, _BW)], uidx_v)
    pltpu.sync_copy(p_idx_hbm.at[pl.ds(base, _BW)], pidx_v)
    pltpu.sync_copy(n_idx_hbm.at[pl.ds(base, _BW)], nidx_v)
    copies = []
    for d in range(_D):
        copies.append(
            pltpu.async_copy(user_t_hbm.at[d].at[uidx_v], u_v.at[d], sem))
        copies.append(
            pltpu.async_copy(item_t_hbm.at[d].at[pidx_v], i_v.at[d], sem))
        copies.append(
            pltpu.async_copy(item_t_hbm.at[d].at[nidx_v], j_v.at[d], sem))
    for c in copies:
        c.wait()

    def group(g, carry):
        r0 = g * 16
        acc = jnp.zeros((16,), jnp.float32)
        for d in range(_D):
            u = u_v[d, pl.ds(r0, 16)]
            i = i_v[d, pl.ds(r0, 16)]
            j = j_v[d, pl.ds(r0, 16)]
            acc = acc + u * (i - j)
        out_v[pl.ds(r0, 16)] = acc
        return carry

    lax.fori_loop(0, _G, group, 0)
    pltpu.sync_copy(out_v, out_hbm.at[pl.ds(base, _BW)])


def kernel(user_input, pos_item_input, neg_item_input, user_emb, item_emb):
    u_idx = user_input.reshape(-1).astype(jnp.int32)
    p_idx = pos_item_input.reshape(-1).astype(jnp.int32)
    n_idx = neg_item_input.reshape(-1).astype(jnp.int32)
    mesh = plsc.VectorSubcoreMesh(core_axis_name="c", subcore_axis_name="s")
    out = pl.kernel(
        _pairwise_body,
        out_type=jax.ShapeDtypeStruct((_B,), jnp.float32),
        mesh=mesh,
        compiler_params=pltpu.CompilerParams(
            needs_layout_passes=False, use_tc_tiling_on_sc=False),
        scratch_types=[
            pltpu.VMEM((_BW,), jnp.int32),
            pltpu.VMEM((_BW,), jnp.int32),
            pltpu.VMEM((_BW,), jnp.int32),
            pltpu.VMEM((_D, _BW), jnp.float32),
            pltpu.VMEM((_D, _BW), jnp.float32),
            pltpu.VMEM((_D, _BW), jnp.float32),
            pltpu.VMEM((_BW,), jnp.float32),
            pltpu.SemaphoreType.DMA,
        ],
    )(u_idx, p_idx, n_idx, user_emb.T, item_emb.T)
    dd = out.reshape(_B, 1)
    return (dd, dd)
